# Initial kernel scaffold; baseline (speedup 1.0000x reference)
import jax
import jax.numpy as jnp
from jax.experimental import pallas as pl


def kernel(x, edge_index, edge_attr, batch_mask, Wrel0, brel0, Wroot0, Wrel1, brel1, Wroot1, Wrel2, brel2, Wroot2):
    raise NotImplementedError("write your pallas kernel here")


# speedup vs baseline: 2.1966x; 2.1966x over previous
import functools

import jax
import jax.numpy as jnp
from jax import lax
from jax.experimental import pallas as pl
from jax.experimental.pallas import tpu as pltpu
from jax.experimental.pallas import tpu_sc as plsc

N = 10000
NP = 10240
E = 160000
DIMS = 4
C = 256
G = 16
EPS = 1e-5

NB = 1024
NBLK = NP // NB

NC = 2
NS = 16
NW = NC * NS
CPT = 4
PASSES = (DIMS * C) // (CPT * NW)
CE = 8000
NCHUNK = E // CE
F32 = jnp.float32


def _transform_body(h_ref, wrel_ref, wroot_ref, b_ref, z_ref, r_ref):
    h = h_ref[0]
    z_ref[0] = jnp.dot(wrel_ref[0], h, preferred_element_type=F32)
    r_ref[0] = jnp.dot(wroot_ref[0], h, preferred_element_type=F32) + b_ref[0, 0][:, None]


def _transform(hT, wrelT, wrootT, b):
    return pl.pallas_call(
        _transform_body,
        grid=(DIMS, NBLK),
        in_specs=[
            pl.BlockSpec((1, C, NB), lambda d, n: (d, 0, n)),
            pl.BlockSpec((1, C, C), lambda d, n: (d, 0, 0)),
            pl.BlockSpec((1, C, C), lambda d, n: (d, 0, 0)),
            pl.BlockSpec((1, 1, C), lambda d, n: (d, 0, 0)),
        ],
        out_specs=[
            pl.BlockSpec((1, C, NB), lambda d, n: (d, 0, n)),
            pl.BlockSpec((1, C, NB), lambda d, n: (d, 0, n)),
        ],
        out_shape=[
            jax.ShapeDtypeStruct((DIMS, C, NP), F32),
            jax.ShapeDtypeStruct((DIMS, C, NP), F32),
        ],
    )(hT, wrelT, wrootT, b)


def _act_stats_body(acc_ref, r_ref, bm_ref, act_ref, s1_ref, s2_ref, cnt_ref):
    d = pl.program_id(0)
    n = pl.program_id(1)
    pre = acc_ref[0] + r_ref[0]
    act = jnp.where(pre >= 0, pre, 0.2 * pre)
    act_ref[0] = act
    bm = bm_ref[0]
    oh = (lax.broadcasted_iota(jnp.int32, (NB, G), 1) == bm[:, None]).astype(F32)
    p1 = jnp.dot(act, oh, preferred_element_type=F32)
    p2 = jnp.dot(act * act, oh, preferred_element_type=F32)

    @pl.when(n == 0)
    def _():
        s1_ref[0] = p1
        s2_ref[0] = p2

    @pl.when(n != 0)
    def _():
        s1_ref[0] += p1
        s2_ref[0] += p2

    pc = jnp.sum(oh, axis=0)[None, :]

    @pl.when((d == 0) & (n == 0))
    def _():
        cnt_ref[...] = pc

    @pl.when((d == 0) & (n != 0))
    def _():
        cnt_ref[...] += pc


def _act_stats(accT, rT, bm2):
    return pl.pallas_call(
        _act_stats_body,
        grid=(DIMS, NBLK),
        in_specs=[
            pl.BlockSpec((1, C, NB), lambda d, n: (d, 0, n)),
            pl.BlockSpec((1, C, NB), lambda d, n: (d, 0, n)),
            pl.BlockSpec((1, NB), lambda d, n: (0, n)),
        ],
        out_specs=[
            pl.BlockSpec((1, C, NB), lambda d, n: (d, 0, n)),
            pl.BlockSpec((1, C, G), lambda d, n: (d, 0, 0)),
            pl.BlockSpec((1, C, G), lambda d, n: (d, 0, 0)),
            pl.BlockSpec((1, G), lambda d, n: (0, 0)),
        ],
        out_shape=[
            jax.ShapeDtypeStruct((DIMS, C, NP), F32),
            jax.ShapeDtypeStruct((DIMS, C, G), F32),
            jax.ShapeDtypeStruct((DIMS, C, G), F32),
            jax.ShapeDtypeStruct((1, G), F32),
        ],
    )(accT, rT, bm2)


def _norm_cols(s1, s2, cnt, bm):
    cnt = jnp.maximum(cnt, 1.0)
    mean = s1 / cnt
    var = jnp.maximum(s2 / cnt - mean * mean, 0.0)
    rstd = lax.rsqrt(var + EPS)
    shift = -mean * rstd
    oht = (lax.broadcasted_iota(jnp.int32, (G, NB), 0) == bm[None, :]).astype(F32)
    scale_c = jnp.dot(rstd, oht, preferred_element_type=F32)
    shift_c = jnp.dot(shift, oht, preferred_element_type=F32)
    return scale_c, shift_c


def _norm_next_body(act_ref, s1_ref, s2_ref, cnt_ref, bm_ref,
                    wrel_ref, wroot_ref, b_ref, out_ref, z_ref, r_ref):
    scale_c, shift_c = _norm_cols(s1_ref[0], s2_ref[0], cnt_ref[...], bm_ref[0])
    norm = act_ref[0] * scale_c + shift_c
    out_ref[0] = norm
    z_ref[0] = jnp.dot(wrel_ref[0], norm, preferred_element_type=F32)
    r_ref[0] = jnp.dot(wroot_ref[0], norm, preferred_element_type=F32) + b_ref[0, 0][:, None]


def _norm_next(act, s1, s2, cnt, bm2, wrelT, wrootT, b):
    return pl.pallas_call(
        _norm_next_body,
        grid=(DIMS, NBLK),
        in_specs=[
            pl.BlockSpec((1, C, NB), lambda d, n: (d, 0, n)),
            pl.BlockSpec((1, C, G), lambda d, n: (d, 0, 0)),
            pl.BlockSpec((1, C, G), lambda d, n: (d, 0, 0)),
            pl.BlockSpec((1, G), lambda d, n: (0, 0)),
            pl.BlockSpec((1, NB), lambda d, n: (0, n)),
            pl.BlockSpec((1, C, C), lambda d, n: (d, 0, 0)),
            pl.BlockSpec((1, C, C), lambda d, n: (d, 0, 0)),
            pl.BlockSpec((1, 1, C), lambda d, n: (d, 0, 0)),
        ],
        out_specs=[
            pl.BlockSpec((1, C, NB), lambda d, n: (d, 0, n)),
            pl.BlockSpec((1, C, NB), lambda d, n: (d, 0, n)),
            pl.BlockSpec((1, C, NB), lambda d, n: (d, 0, n)),
        ],
        out_shape=[
            jax.ShapeDtypeStruct((DIMS, C, NP), F32),
            jax.ShapeDtypeStruct((DIMS, C, NP), F32),
            jax.ShapeDtypeStruct((DIMS, C, NP), F32),
        ],
    )(act, s1, s2, cnt, bm2, wrelT, wrootT, b)


def _norm_final_body(act_ref, s1_ref, s2_ref, cnt_ref, bm_ref, out_ref):
    scale_c, shift_c = _norm_cols(s1_ref[0], s2_ref[0], cnt_ref[...], bm_ref[0])
    out_ref[0] = act_ref[0] * scale_c + shift_c


def _norm_final(act, s1, s2, cnt, bm2):
    return pl.pallas_call(
        _norm_final_body,
        grid=(DIMS, NBLK),
        in_specs=[
            pl.BlockSpec((1, C, NB), lambda d, n: (d, 0, n)),
            pl.BlockSpec((1, C, G), lambda d, n: (d, 0, 0)),
            pl.BlockSpec((1, C, G), lambda d, n: (d, 0, 0)),
            pl.BlockSpec((1, G), lambda d, n: (0, 0)),
            pl.BlockSpec((1, NB), lambda d, n: (0, n)),
        ],
        out_specs=[pl.BlockSpec((1, C, NB), lambda d, n: (d, 0, n))],
        out_shape=[jax.ShapeDtypeStruct((DIMS, C, NP), F32)],
    )(act, s1, s2, cnt, bm2)


def _sc_msg_body(z_hbm, pk_hbm, ew_hbm, acc_hbm,
                 zsem, esem0, esem1, zv, av, pk0, pk1, ew0, ew1):
    cid = lax.axis_index("c")
    sid = lax.axis_index("s")
    wid = sid * NC + cid

    def pass_body(p, _):
        item = wid * PASSES + p
        d = lax.shift_right_logical(item, 6)
        coff = d * (C * NP) + (item & 63) * (CPT * NP)
        eoff = d * E

        pltpu.async_copy(z_hbm.at[pl.ds(coff, CPT * NP)], zv, zsem).wait()

        zero = jnp.zeros((16,), F32)

        @pl.loop(0, (CPT * NP) // 16, unroll=8)
        def _(i):
            av[pl.ds(i * 16, 16)] = zero

        def issue(g, pkb, ewb, sem):
            cp = pltpu.async_copy(pk_hbm.at[pl.ds(g * CE, CE)], pkb, sem)
            ce_ = pltpu.async_copy(ew_hbm.at[pl.ds(eoff + g * CE, CE)], ewb, sem)
            return cp, ce_

        def consume(pkb, ewb):
            @pl.loop(0, CE // 16, unroll=2)
            def _(i):
                base = i * 16
                pk = pkb[pl.ds(base, 16)]
                ew = ewb[pl.ds(base, 16)]
                srcv = pk & 0xFFFF
                dstv = lax.shift_right_logical(pk, 16)
                for j in range(CPT):
                    g16 = plsc.load_gather(zv, [srcv + (j * NP)])
                    plsc.addupdate_scatter(av, [dstv + (j * NP)], g16 * ew)

        bufs = ((pk0, ew0, esem0), (pk1, ew1, esem1))
        pend = [issue(0, *bufs[0])]
        for g in range(NCHUNK):
            if g + 1 < NCHUNK:
                pend.append(issue(g + 1, *bufs[(g + 1) % 2]))
            for desc in pend.pop(0):
                desc.wait()
            consume(bufs[g % 2][0], bufs[g % 2][1])

        pltpu.sync_copy(av, acc_hbm.at[pl.ds(coff, CPT * NP)])
        return 0

    lax.fori_loop(0, PASSES, pass_body, 0)


@functools.lru_cache(maxsize=None)
def _build_sc_message():
    return functools.partial(
        pl.kernel,
        out_type=jax.ShapeDtypeStruct((DIMS * C * NP,), F32),
        mesh=plsc.VectorSubcoreMesh(core_axis_name="c", subcore_axis_name="s",
                                    num_cores=NC, num_subcores=NS),
        compiler_params=pltpu.CompilerParams(needs_layout_passes=False),
        scratch_types=[
            pltpu.SemaphoreType.DMA,
            pltpu.SemaphoreType.DMA,
            pltpu.SemaphoreType.DMA,
            pltpu.VMEM((CPT * NP,), F32),
            pltpu.VMEM((CPT * NP,), F32),
            pltpu.VMEM((CE,), jnp.int32),
            pltpu.VMEM((CE,), jnp.int32),
            pltpu.VMEM((CE,), F32),
            pltpu.VMEM((CE,), F32),
        ],
    )(_sc_msg_body)


def _sc_message(z2d, packed, ewT):
    return _build_sc_message()(z2d.reshape(-1), packed, ewT.reshape(-1))


def kernel(x, edge_index, edge_attr, batch_mask,
           Wrel0, brel0, Wroot0, Wrel1, brel1, Wroot1, Wrel2, brel2, Wroot2):
    hT = x.reshape(N, C, DIMS).transpose(2, 1, 0)
    hT = jnp.pad(hT, ((0, 0), (0, 0), (0, NP - N)))
    bm2 = jnp.pad(batch_mask, (0, NP - N), constant_values=G).reshape(1, NP)

    src = edge_index[0]
    dst = edge_index[1]
    packed = jnp.bitwise_or(jnp.left_shift(dst, 16), src)
    ewT = edge_attr.T

    fold = lambda W: W[:, :C, :] + W[:, C:, :]
    wrelT = [Wrel0.transpose(0, 2, 1),
             fold(Wrel1).transpose(0, 2, 1),
             fold(Wrel2).transpose(0, 2, 1)]
    wrootT = [Wroot0.transpose(0, 2, 1),
              fold(Wroot1).transpose(0, 2, 1),
              fold(Wroot2).transpose(0, 2, 1)]
    brel = [b.reshape(DIMS, 1, C) for b in (brel0, brel1, brel2)]

    zT, rT = _transform(hT, wrelT[0], wrootT[0], brel[0])
    outs = []
    for l in range(3):
        accT = _sc_message(zT, packed, ewT).reshape(DIMS, C, NP)
        act, s1, s2, cnt = _act_stats(accT, rT, bm2)
        if l < 2:
            outl, zT, rT = _norm_next(act, s1, s2, cnt, bm2,
                                      wrelT[l + 1], wrootT[l + 1], brel[l + 1])
        else:
            (outl,) = _norm_final(act, s1, s2, cnt, bm2)
        outs.append(outl)

    out = jnp.stack(outs, axis=1)
    return out[:, :, :, :N].transpose(0, 1, 3, 2)
